# baseline (speedup 1.0000x reference)
import functools

import jax
import jax.numpy as jnp
import numpy as np
from jax.experimental import pallas as pl
from jax.experimental.pallas import tpu as pltpu

F32 = jnp.float32
BF16 = jnp.bfloat16

_CP = pltpu.CompilerParams(vmem_limit_bytes=100 * 1024 * 1024)


def _fps_body(npoint, n, xc_ref, yc_ref, zc_ref, newxyz_ref):
    b = xc_ref.shape[0]
    r = n // 128
    lin = (jax.lax.broadcasted_iota(jnp.int32, (1, r, 128), 1) * 128
           + jax.lax.broadcasted_iota(jnp.int32, (1, r, 128), 2))
    xc = xc_ref[...]
    yc = yc_ref[...]
    zc = zc_ref[...]

    def red_sum(a):
        return jnp.sum(jnp.sum(a, axis=2, keepdims=True), axis=1, keepdims=True)

    def red_max(a):
        return jnp.max(jnp.max(a, axis=2, keepdims=True), axis=1, keepdims=True)

    def red_min(a):
        return jnp.min(jnp.min(a, axis=2, keepdims=True), axis=1, keepdims=True)

    def body(i, carry):
        dist, f = carry
        onehot = lin == f
        cx = red_sum(jnp.where(onehot, xc, 0.0))
        cy = red_sum(jnp.where(onehot, yc, 0.0))
        cz = red_sum(jnp.where(onehot, zc, 0.0))
        for bb in range(b):
            newxyz_ref[bb, pl.ds(i, 1), 0:1] = cx[bb]
            newxyz_ref[bb, pl.ds(i, 1), 1:2] = cy[bb]
            newxyz_ref[bb, pl.ds(i, 1), 2:3] = cz[bb]
        dx = xc - cx
        dy = yc - cy
        dz = zc - cz
        d = (dx * dx + dy * dy) + dz * dz
        dist = jnp.minimum(dist, d)
        m = red_max(dist)
        fn = red_min(jnp.where(dist >= m, lin, n))
        return dist, fn

    dist0 = jnp.full((b, r, 128), 1e10, F32)
    f0 = jnp.zeros((b, 1, 1), jnp.int32)
    jax.lax.fori_loop(0, npoint, body, (dist0, f0))


def _fps(xyz_pts, npoint):
    b, n, _ = xyz_pts.shape
    r = n // 128
    ch = jnp.transpose(xyz_pts, (0, 2, 1)).reshape(b, 3, r, 128)
    return pl.pallas_call(
        functools.partial(_fps_body, npoint, n),
        out_shape=jax.ShapeDtypeStruct((b, npoint, 3), F32),
        compiler_params=_CP,
    )(ch[:, 0], ch[:, 1], ch[:, 2])


def _sa_body(pu_ref, xyzT_ref, cxyz_ref, tri_ref, wu_ref, wvx_ref, b1_ref,
             w2_ref, b2_ref, w3_ref, b3_ref, out_ref, ucat_ref,
             *, r2, K, Kc, TS, n, C1, C3):
    t = pl.program_id(1)

    @pl.when(t == 0)
    def _():
        u = jnp.dot(pu_ref[0], wu_ref[...], preferred_element_type=F32)
        ucat_ref[...] = u.astype(BF16)

    cs = cxyz_ref[0]
    xT = xyzT_ref[0]
    cn = (cs[:, 0:1] * cs[:, 0:1] + cs[:, 1:2] * cs[:, 1:2]) + cs[:, 2:3] * cs[:, 2:3]
    xn = (xT[0:1] * xT[0:1] + xT[1:2] * xT[1:2]) + xT[2:3] * xT[2:3]
    dotp = jnp.dot(cs.astype(BF16), xT.astype(BF16), preferred_element_type=F32)
    d = (cn + xn) - 2.0 * dotp
    mask = d <= r2
    rank = jnp.dot(mask.astype(BF16), tri_ref[...], preferred_element_type=F32)
    count = jnp.sum(mask.astype(F32), axis=1, keepdims=True)
    rankk = jnp.where(mask, rank, -1.0)
    lastcol = jax.lax.broadcasted_iota(jnp.int32, (1, n), 1) == (n - 1)
    rankk = jnp.where(jnp.logical_and(count == 0.0, lastcol), 0.0, rankk)
    count = jnp.maximum(count, 1.0)
    rankb = rankk.astype(BF16)
    vterm = b1_ref[...] - jnp.dot(cs, wvx_ref[...], preferred_element_type=F32)
    ucat = ucat_ref[...]
    w2 = w2_ref[...].astype(BF16)
    b2 = b2_ref[...]
    w3 = w3_ref[...].astype(BF16)
    b3 = b3_ref[...]

    def chunk(kc, acc):
        kio = jax.lax.broadcasted_iota(jnp.int32, (1, Kc, 1), 1) + kc * Kc
        kiof = kio.astype(F32)
        psel = (rankb[:, None, :] == kio.astype(BF16)).astype(BF16)
        f1 = jnp.dot(psel.reshape(TS * Kc, n), ucat, preferred_element_type=F32)
        h = jnp.maximum(f1.reshape(TS, Kc, C1) + vterm[:, None, :], 0.0)
        h = jnp.maximum(jnp.dot(h.reshape(TS * Kc, C1).astype(BF16), w2,
                                preferred_element_type=F32) + b2, 0.0)
        h = jnp.maximum(jnp.dot(h.astype(BF16), w3,
                                preferred_element_type=F32) + b3, 0.0)
        valid = kiof < count[:, :, None]
        hm = jnp.where(valid, h.reshape(TS, Kc, C3), 0.0)
        return jnp.maximum(acc, jnp.max(hm, axis=1))

    out_ref[0] = jax.lax.fori_loop(0, K // Kc, chunk, jnp.zeros((TS, C3), F32))


def _sa_branch(pu, xyzT, cxyz, radius, K, TS, Kc, layers):
    b, n, cpu = pu.shape
    s = cxyz.shape[1]
    wu, b1 = _fuse(layers[0])
    w2, b2 = _fuse(layers[1])
    w3, b3 = _fuse(layers[2])
    c1, c3 = wu.shape[1], w3.shape[1]
    wvx = wu[cpu - 3:]
    tri = (jax.lax.broadcasted_iota(jnp.int32, (n, n), 0)
           < jax.lax.broadcasted_iota(jnp.int32, (n, n), 1)).astype(BF16)
    r2 = float(np.float32(radius ** 2))
    body = functools.partial(_sa_body, r2=r2, K=K, Kc=Kc, TS=TS, n=n,
                             C1=c1, C3=c3)
    return pl.pallas_call(
        body,
        grid=(b, s // TS),
        in_specs=[
            pl.BlockSpec((1, n, cpu), lambda bi, ti: (bi, 0, 0)),
            pl.BlockSpec((1, 3, n), lambda bi, ti: (bi, 0, 0)),
            pl.BlockSpec((1, TS, 3), lambda bi, ti: (bi, ti, 0)),
            pl.BlockSpec((n, n), lambda bi, ti: (0, 0)),
            pl.BlockSpec(wu.shape, lambda bi, ti: (0, 0)),
            pl.BlockSpec(wvx.shape, lambda bi, ti: (0, 0)),
            pl.BlockSpec(b1.shape, lambda bi, ti: (0, 0)),
            pl.BlockSpec(w2.shape, lambda bi, ti: (0, 0)),
            pl.BlockSpec(b2.shape, lambda bi, ti: (0, 0)),
            pl.BlockSpec(w3.shape, lambda bi, ti: (0, 0)),
            pl.BlockSpec(b3.shape, lambda bi, ti: (0, 0)),
        ],
        out_specs=pl.BlockSpec((1, TS, c3), lambda bi, ti: (bi, ti, 0)),
        out_shape=jax.ShapeDtypeStruct((b, s, c3), F32),
        scratch_shapes=[pltpu.VMEM((n, c1), BF16)],
        compiler_params=_CP,
    )(pu, xyzT, cxyz, tri, wu, wvx, b1, w2, b2, w3, b3)


def _fp_body(x1_ref, x2T_ref, p1_ref, p2_ref, w1a_ref, w1b_ref, b1_ref,
             w2_ref, b2_ref, out_ref, *, S2):
    cs = x1_ref[0]
    xT = x2T_ref[0]
    cn = (cs[:, 0:1] * cs[:, 0:1] + cs[:, 1:2] * cs[:, 1:2]) + cs[:, 2:3] * cs[:, 2:3]
    xn = (xT[0:1] * xT[0:1] + xT[1:2] * xT[1:2]) + xT[2:3] * xT[2:3]
    dotp = jnp.dot(cs.astype(BF16), xT.astype(BF16), preferred_element_type=F32)
    d = (cn + xn) - 2.0 * dotp
    io = jax.lax.broadcasted_iota(jnp.int32, (1, S2), 1)
    inf = jnp.float32(float("inf"))
    m1 = jnp.min(d, axis=1, keepdims=True)
    i1 = jnp.min(jnp.where(d <= m1, io, S2), axis=1, keepdims=True)
    d1 = jnp.where(io == i1, inf, d)
    m2 = jnp.min(d1, axis=1, keepdims=True)
    i2 = jnp.min(jnp.where(d1 <= m2, io, S2), axis=1, keepdims=True)
    d2 = jnp.where(io == i2, inf, d1)
    m3 = jnp.min(d2, axis=1, keepdims=True)
    i3 = jnp.min(jnp.where(d2 <= m3, io, S2), axis=1, keepdims=True)
    r1 = 1.0 / (m1 + 1e-8)
    r2 = 1.0 / (m2 + 1e-8)
    r3 = 1.0 / (m3 + 1e-8)
    s = (r1 + r2) + r3
    W = (jnp.where(io == i1, r1 / s, 0.0)
         + jnp.where(io == i2, r2 / s, 0.0)
         + jnp.where(io == i3, r3 / s, 0.0))
    interp = jnp.dot(W, p2_ref[0], preferred_element_type=F32)
    h = jnp.maximum(jnp.dot(p1_ref[0], w1a_ref[...], preferred_element_type=F32)
                    + jnp.dot(interp, w1b_ref[...], preferred_element_type=F32)
                    + b1_ref[...], 0.0)
    out_ref[0] = jnp.maximum(jnp.dot(h, w2_ref[...], preferred_element_type=F32)
                             + b2_ref[...], 0.0)


def _fp(x1, x2, p1, p2, TN, layers):
    b, n1, cp1 = p1.shape
    s2, cp2 = p2.shape[1], p2.shape[2]
    wf1, b1 = _fuse(layers[0])
    w2, b2 = _fuse(layers[1])
    w1a, w1b = wf1[:cp1], wf1[cp1:]
    c2 = w2.shape[1]
    x2T = jnp.transpose(x2, (0, 2, 1))
    return pl.pallas_call(
        functools.partial(_fp_body, S2=s2),
        grid=(b, n1 // TN),
        in_specs=[
            pl.BlockSpec((1, TN, 3), lambda bi, ti: (bi, ti, 0)),
            pl.BlockSpec((1, 3, s2), lambda bi, ti: (bi, 0, 0)),
            pl.BlockSpec((1, TN, cp1), lambda bi, ti: (bi, ti, 0)),
            pl.BlockSpec((1, s2, cp2), lambda bi, ti: (bi, 0, 0)),
            pl.BlockSpec(w1a.shape, lambda bi, ti: (0, 0)),
            pl.BlockSpec(w1b.shape, lambda bi, ti: (0, 0)),
            pl.BlockSpec(b1.shape, lambda bi, ti: (0, 0)),
            pl.BlockSpec(w2.shape, lambda bi, ti: (0, 0)),
            pl.BlockSpec(b2.shape, lambda bi, ti: (0, 0)),
        ],
        out_specs=pl.BlockSpec((1, TN, c2), lambda bi, ti: (bi, ti, 0)),
        out_shape=jax.ShapeDtypeStruct((b, n1, c2), F32),
        compiler_params=_CP,
    )(x1, x2T, p1, p2, w1a, w1b, b1, w2, b2)


def _sa3_body(x2_ref, p2_ref, w1a_ref, w1b_ref, b1_ref, w2_ref, b2_ref,
              w3_ref, b3_ref, out_ref):
    h = jnp.maximum(jnp.dot(x2_ref[0], w1a_ref[...], preferred_element_type=F32)
                    + jnp.dot(p2_ref[0], w1b_ref[...], preferred_element_type=F32)
                    + b1_ref[...], 0.0)
    h = jnp.maximum(jnp.dot(h, w2_ref[...], preferred_element_type=F32) + b2_ref[...], 0.0)
    h = jnp.maximum(jnp.dot(h, w3_ref[...], preferred_element_type=F32) + b3_ref[...], 0.0)
    out_ref[0] = jnp.max(h, axis=0, keepdims=True)


def _sa3(l2_xyz, l2_points, layers):
    b, s, cp = l2_points.shape
    wf1, b1 = _fuse(layers[0])
    w2, b2 = _fuse(layers[1])
    w3, b3 = _fuse(layers[2])
    w1a, w1b = wf1[:3], wf1[3:]
    c3 = w3.shape[1]
    return pl.pallas_call(
        _sa3_body,
        grid=(b,),
        in_specs=[
            pl.BlockSpec((1, s, 3), lambda bi: (bi, 0, 0)),
            pl.BlockSpec((1, s, cp), lambda bi: (bi, 0, 0)),
            pl.BlockSpec(w1a.shape, lambda bi: (0, 0)),
            pl.BlockSpec(w1b.shape, lambda bi: (0, 0)),
            pl.BlockSpec(b1.shape, lambda bi: (0, 0)),
            pl.BlockSpec(w2.shape, lambda bi: (0, 0)),
            pl.BlockSpec(b2.shape, lambda bi: (0, 0)),
            pl.BlockSpec(w3.shape, lambda bi: (0, 0)),
            pl.BlockSpec(b3.shape, lambda bi: (0, 0)),
        ],
        out_specs=pl.BlockSpec((1, 1, c3), lambda bi: (bi, 0, 0)),
        out_shape=jax.ShapeDtypeStruct((b, 1, c3), F32),
        compiler_params=_CP,
    )(l2_xyz, l2_points, w1a, w1b, b1, w2, b2, w3, b3)


def _fp3_body(p1_ref, l3_ref, w1a_ref, w1b_ref, b1_ref, w2_ref, b2_ref, out_ref):
    base = jnp.dot(l3_ref[0], w1b_ref[...], preferred_element_type=F32)
    h = jnp.maximum(jnp.dot(p1_ref[0], w1a_ref[...], preferred_element_type=F32)
                    + base + b1_ref[...], 0.0)
    out_ref[0] = jnp.maximum(jnp.dot(h, w2_ref[...], preferred_element_type=F32)
                             + b2_ref[...], 0.0)


def _fp3(l2_points, l3_points, layers):
    b, s, cp = l2_points.shape
    cl3 = l3_points.shape[2]
    wf1, b1 = _fuse(layers[0])
    w2, b2 = _fuse(layers[1])
    w1a, w1b = wf1[:cp], wf1[cp:]
    c2 = w2.shape[1]
    return pl.pallas_call(
        _fp3_body,
        grid=(b,),
        in_specs=[
            pl.BlockSpec((1, s, cp), lambda bi: (bi, 0, 0)),
            pl.BlockSpec((1, 1, cl3), lambda bi: (bi, 0, 0)),
            pl.BlockSpec(w1a.shape, lambda bi: (0, 0)),
            pl.BlockSpec(w1b.shape, lambda bi: (0, 0)),
            pl.BlockSpec(b1.shape, lambda bi: (0, 0)),
            pl.BlockSpec(w2.shape, lambda bi: (0, 0)),
            pl.BlockSpec(b2.shape, lambda bi: (0, 0)),
        ],
        out_specs=pl.BlockSpec((1, s, c2), lambda bi: (bi, 0, 0)),
        out_shape=jax.ShapeDtypeStruct((b, s, c2), F32),
        compiler_params=_CP,
    )(l2_points, l3_points, w1a, w1b, b1, w2, b2)


def _head_body(x_ref, w1_ref, b1_ref, w2_ref, b2_ref, out_ref):
    feat = jnp.maximum(jnp.dot(x_ref[0], w1_ref[...], preferred_element_type=F32)
                       + b1_ref[...], 0.0)
    logits = jnp.dot(feat, w2_ref[...], preferred_element_type=F32) + b2_ref[...]
    m = jnp.max(logits, axis=1, keepdims=True)
    sh = logits - m
    out_ref[0] = sh - jnp.log(jnp.sum(jnp.exp(sh), axis=1, keepdims=True))


def _head(l0_fp, conv1, conv2):
    b, n, _ = l0_fp.shape
    w1, b1 = _fuse(conv1)
    w2 = jnp.transpose(conv2['w'])
    b2 = conv2['b'][None, :]
    nc = w2.shape[1]
    return pl.pallas_call(
        _head_body,
        grid=(b,),
        in_specs=[
            pl.BlockSpec((1, n, l0_fp.shape[2]), lambda bi: (bi, 0, 0)),
            pl.BlockSpec(w1.shape, lambda bi: (0, 0)),
            pl.BlockSpec(b1.shape, lambda bi: (0, 0)),
            pl.BlockSpec(w2.shape, lambda bi: (0, 0)),
            pl.BlockSpec(b2.shape, lambda bi: (0, 0)),
        ],
        out_specs=pl.BlockSpec((1, n, nc), lambda bi: (bi, 0, 0)),
        out_shape=jax.ShapeDtypeStruct((b, n, nc), F32),
        compiler_params=_CP,
    )(l0_fp, w1, b1, w2, b2)


def _fuse(layer):
    w = jnp.transpose(layer['w']) * layer['gamma'][None, :]
    bias = (layer['b'] * layer['gamma'] + layer['beta'])[None, :]
    return w, bias


def kernel(xyz, cls_label, params):
    b, _, n = xyz.shape
    l0_xyz = jnp.transpose(xyz, (0, 2, 1))

    l1_xyz = _fps(l0_xyz, 1024)
    pu1 = jnp.concatenate([l0_xyz, l0_xyz], axis=-1)
    outs = []
    for radius, K, layers in zip([0.1, 0.2, 0.4], [32, 64, 128], params['sa1']):
        outs.append(_sa_branch(pu1, xyz, l1_xyz, radius, K, 256, 4, layers))
    l1_points = jnp.concatenate(outs, axis=-1)

    l2_xyz = _fps(l1_xyz, 128)
    pu2 = jnp.concatenate([l1_points, l1_xyz], axis=-1)
    xyzT1 = jnp.transpose(l1_xyz, (0, 2, 1))
    outs2 = []
    for radius, K, layers in zip([0.4, 0.8], [64, 128], params['sa2']):
        outs2.append(_sa_branch(pu2, xyzT1, l2_xyz, radius, K, 128, 8, layers))
    l2_points = jnp.concatenate(outs2, axis=-1)

    l3_points = _sa3(l2_xyz, l2_points, params['sa3'])
    l2_fp = _fp3(l2_points, l3_points, params['fp3'])
    l1_fp = _fp(l1_xyz, l2_xyz, l1_points, l2_fp, 256, params['fp2'])
    cls_one = jnp.broadcast_to(cls_label.reshape(b, 1, 1), (b, n, 1))
    p1 = jnp.concatenate([cls_one, l0_xyz, l0_xyz], axis=-1)
    l0_fp = _fp(l0_xyz, l1_xyz, p1, l1_fp, 256, params['fp1'])

    x = _head(l0_fp, params['conv1'], params['conv2'])
    return x, jnp.transpose(l3_points, (0, 2, 1))


# speedup vs baseline: 1.2739x; 1.2739x over previous
import functools

import jax
import jax.numpy as jnp
import numpy as np
from jax.experimental import pallas as pl
from jax.experimental.pallas import tpu as pltpu

F32 = jnp.float32
BF16 = jnp.bfloat16

_CP = pltpu.CompilerParams(vmem_limit_bytes=100 * 1024 * 1024)


def _fps_body(npoint, n, xc_ref, yc_ref, zc_ref, newxyz_ref):
    b = xc_ref.shape[0]
    r = n // 128
    lin = (jax.lax.broadcasted_iota(jnp.int32, (1, r, 128), 1) * 128
           + jax.lax.broadcasted_iota(jnp.int32, (1, r, 128), 2))
    xc = xc_ref[...]
    yc = yc_ref[...]
    zc = zc_ref[...]

    def red_sum(a):
        return jnp.sum(jnp.sum(a, axis=2, keepdims=True), axis=1, keepdims=True)

    def red_max(a):
        return jnp.max(jnp.max(a, axis=2, keepdims=True), axis=1, keepdims=True)

    def red_min(a):
        return jnp.min(jnp.min(a, axis=2, keepdims=True), axis=1, keepdims=True)

    def body(i, carry):
        dist, f = carry
        onehot = lin == f
        cx = red_sum(jnp.where(onehot, xc, 0.0))
        cy = red_sum(jnp.where(onehot, yc, 0.0))
        cz = red_sum(jnp.where(onehot, zc, 0.0))
        for bb in range(b):
            newxyz_ref[bb, pl.ds(i, 1), 0:1] = cx[bb]
            newxyz_ref[bb, pl.ds(i, 1), 1:2] = cy[bb]
            newxyz_ref[bb, pl.ds(i, 1), 2:3] = cz[bb]
        dx = xc - cx
        dy = yc - cy
        dz = zc - cz
        d = (dx * dx + dy * dy) + dz * dz
        dist = jnp.minimum(dist, d)
        m = red_max(dist)
        fn = red_min(jnp.where(dist >= m, lin, n))
        return dist, fn

    dist0 = jnp.full((b, r, 128), 1e10, F32)
    f0 = jnp.zeros((b, 1, 1), jnp.int32)
    jax.lax.fori_loop(0, npoint, body, (dist0, f0))


def _fps(xyz_pts, npoint):
    b, n, _ = xyz_pts.shape
    r = n // 128
    ch = jnp.transpose(xyz_pts, (0, 2, 1)).reshape(b, 3, r, 128)
    return pl.pallas_call(
        functools.partial(_fps_body, npoint, n),
        out_shape=jax.ShapeDtypeStruct((b, npoint, 3), F32),
        compiler_params=_CP,
    )(ch[:, 0], ch[:, 1], ch[:, 2])


_CW = 256


def _sa_body(pu_ref, xyzT_ref, cxyz_ref, tri_ref, wu_ref, wvx_ref, b1_ref,
             w2_ref, b2_ref, w3_ref, b3_ref, out_ref, ucat_ref, rankb_ref,
             *, r2, K, Kc, TS, n, C1, C3):
    t = pl.program_id(1)

    @pl.when(t == 0)
    def _():
        u = jnp.dot(pu_ref[0], wu_ref[...], preferred_element_type=F32)
        ucat_ref[...] = u.astype(BF16)

    cs = cxyz_ref[0]
    xT = xyzT_ref[0]
    cn = (cs[:, 0:1] * cs[:, 0:1] + cs[:, 1:2] * cs[:, 1:2]) + cs[:, 2:3] * cs[:, 2:3]
    xn = (xT[0:1] * xT[0:1] + xT[1:2] * xT[1:2]) + xT[2:3] * xT[2:3]
    dotp = jnp.dot(cs.astype(BF16), xT.astype(BF16), preferred_element_type=F32)
    d = (cn + xn) - 2.0 * dotp
    mask = d <= r2
    rank = jnp.dot(mask.astype(BF16), tri_ref[...], preferred_element_type=F32)
    count = jnp.sum(mask.astype(F32), axis=1, keepdims=True)
    rankk = jnp.where(mask, rank, -1.0)
    lastcol = jax.lax.broadcasted_iota(jnp.int32, (1, n), 1) == (n - 1)
    rankk = jnp.where(jnp.logical_and(count == 0.0, lastcol), 0.0, rankk)
    count = jnp.maximum(count, 1.0)
    for c in range(n // _CW):
        rankb_ref[c] = rankk[:, c * _CW:(c + 1) * _CW]
    vterm = b1_ref[...] - jnp.dot(cs, wvx_ref[...], preferred_element_type=F32)
    w2 = w2_ref[...]
    b2 = b2_ref[...]
    w3 = w3_ref[...]
    b3 = b3_ref[...]

    nchunks = jnp.ceil(jnp.minimum(jnp.max(count), float(K)) / Kc).astype(jnp.int32)
    io_n = jax.lax.broadcasted_iota(jnp.int32, (1, n), 1)
    colk = jnp.min(jnp.where(rank >= float(K), io_n, n), axis=1, keepdims=True)
    ncc = jnp.minimum((jnp.max(colk) + (_CW - 1)) // _CW, n // _CW)

    def chunk(kc, acc):
        kio = jax.lax.broadcasted_iota(jnp.int32, (1, Kc, 1), 1) + kc * Kc
        kiof = kio.astype(F32)

        def colchunk(cc, f1):
            pselc = (rankb_ref[cc][:, None, :] == kiof).astype(BF16)
            f1c = jnp.dot(pselc.reshape(TS * Kc, _CW),
                          ucat_ref[pl.ds(pl.multiple_of(cc * _CW, _CW), _CW), :],
                          preferred_element_type=F32)
            return f1 + f1c

        f1 = jax.lax.fori_loop(0, ncc, colchunk,
                               jnp.zeros((TS * Kc, C1), F32))
        h = jnp.maximum(f1.reshape(TS, Kc, C1) + vterm[:, None, :], 0.0)
        h = jnp.maximum(jnp.dot(h.reshape(TS * Kc, C1), w2,
                                preferred_element_type=F32) + b2, 0.0)
        h = jnp.maximum(jnp.dot(h, w3,
                                preferred_element_type=F32) + b3, 0.0)
        valid = kiof < count[:, :, None]
        hm = jnp.where(valid, h.reshape(TS, Kc, C3), 0.0)
        return jnp.maximum(acc, jnp.max(hm, axis=1))

    out_ref[0] = jax.lax.fori_loop(0, nchunks, chunk, jnp.zeros((TS, C3), F32))


def _sa_branch(pu, xyzT, cxyz, radius, K, TS, Kc, layers):
    b, n, cpu = pu.shape
    s = cxyz.shape[1]
    wu, b1 = _fuse(layers[0])
    w2, b2 = _fuse(layers[1])
    w3, b3 = _fuse(layers[2])
    c1, c3 = wu.shape[1], w3.shape[1]
    wvx = wu[cpu - 3:]
    tri = (jax.lax.broadcasted_iota(jnp.int32, (n, n), 0)
           < jax.lax.broadcasted_iota(jnp.int32, (n, n), 1)).astype(BF16)
    r2 = float(np.float32(radius ** 2))
    body = functools.partial(_sa_body, r2=r2, K=K, Kc=Kc, TS=TS, n=n,
                             C1=c1, C3=c3)
    return pl.pallas_call(
        body,
        grid=(b, s // TS),
        in_specs=[
            pl.BlockSpec((1, n, cpu), lambda bi, ti: (bi, 0, 0)),
            pl.BlockSpec((1, 3, n), lambda bi, ti: (bi, 0, 0)),
            pl.BlockSpec((1, TS, 3), lambda bi, ti: (bi, ti, 0)),
            pl.BlockSpec((n, n), lambda bi, ti: (0, 0)),
            pl.BlockSpec(wu.shape, lambda bi, ti: (0, 0)),
            pl.BlockSpec(wvx.shape, lambda bi, ti: (0, 0)),
            pl.BlockSpec(b1.shape, lambda bi, ti: (0, 0)),
            pl.BlockSpec(w2.shape, lambda bi, ti: (0, 0)),
            pl.BlockSpec(b2.shape, lambda bi, ti: (0, 0)),
            pl.BlockSpec(w3.shape, lambda bi, ti: (0, 0)),
            pl.BlockSpec(b3.shape, lambda bi, ti: (0, 0)),
        ],
        out_specs=pl.BlockSpec((1, TS, c3), lambda bi, ti: (bi, ti, 0)),
        out_shape=jax.ShapeDtypeStruct((b, s, c3), F32),
        scratch_shapes=[pltpu.VMEM((n, c1), BF16),
                        pltpu.VMEM((n // _CW, TS, _CW), F32)],
        compiler_params=_CP,
    )(pu, xyzT, cxyz, tri, wu, wvx, b1, w2, b2, w3, b3)


def _fp_body(x1_ref, x2T_ref, p1_ref, p2_ref, w1a_ref, w1b_ref, b1_ref,
             w2_ref, b2_ref, out_ref, *, S2):
    cs = x1_ref[0]
    xT = x2T_ref[0]
    cn = (cs[:, 0:1] * cs[:, 0:1] + cs[:, 1:2] * cs[:, 1:2]) + cs[:, 2:3] * cs[:, 2:3]
    xn = (xT[0:1] * xT[0:1] + xT[1:2] * xT[1:2]) + xT[2:3] * xT[2:3]
    dotp = jnp.dot(cs.astype(BF16), xT.astype(BF16), preferred_element_type=F32)
    d = (cn + xn) - 2.0 * dotp
    io = jax.lax.broadcasted_iota(jnp.int32, (1, S2), 1)
    inf = jnp.float32(float("inf"))
    m1 = jnp.min(d, axis=1, keepdims=True)
    i1 = jnp.min(jnp.where(d <= m1, io, S2), axis=1, keepdims=True)
    d1 = jnp.where(io == i1, inf, d)
    m2 = jnp.min(d1, axis=1, keepdims=True)
    i2 = jnp.min(jnp.where(d1 <= m2, io, S2), axis=1, keepdims=True)
    d2 = jnp.where(io == i2, inf, d1)
    m3 = jnp.min(d2, axis=1, keepdims=True)
    i3 = jnp.min(jnp.where(d2 <= m3, io, S2), axis=1, keepdims=True)
    r1 = 1.0 / (m1 + 1e-8)
    r2 = 1.0 / (m2 + 1e-8)
    r3 = 1.0 / (m3 + 1e-8)
    s = (r1 + r2) + r3
    W = (jnp.where(io == i1, r1 / s, 0.0)
         + jnp.where(io == i2, r2 / s, 0.0)
         + jnp.where(io == i3, r3 / s, 0.0))
    interp = jnp.dot(W, p2_ref[0], preferred_element_type=F32)
    h = jnp.maximum(jnp.dot(p1_ref[0], w1a_ref[...], preferred_element_type=F32)
                    + jnp.dot(interp, w1b_ref[...], preferred_element_type=F32)
                    + b1_ref[...], 0.0)
    out_ref[0] = jnp.maximum(jnp.dot(h, w2_ref[...], preferred_element_type=F32)
                             + b2_ref[...], 0.0)


def _fp(x1, x2, p1, p2, TN, layers):
    b, n1, cp1 = p1.shape
    s2, cp2 = p2.shape[1], p2.shape[2]
    wf1, b1 = _fuse(layers[0])
    w2, b2 = _fuse(layers[1])
    w1a, w1b = wf1[:cp1], wf1[cp1:]
    c2 = w2.shape[1]
    x2T = jnp.transpose(x2, (0, 2, 1))
    return pl.pallas_call(
        functools.partial(_fp_body, S2=s2),
        grid=(b, n1 // TN),
        in_specs=[
            pl.BlockSpec((1, TN, 3), lambda bi, ti: (bi, ti, 0)),
            pl.BlockSpec((1, 3, s2), lambda bi, ti: (bi, 0, 0)),
            pl.BlockSpec((1, TN, cp1), lambda bi, ti: (bi, ti, 0)),
            pl.BlockSpec((1, s2, cp2), lambda bi, ti: (bi, 0, 0)),
            pl.BlockSpec(w1a.shape, lambda bi, ti: (0, 0)),
            pl.BlockSpec(w1b.shape, lambda bi, ti: (0, 0)),
            pl.BlockSpec(b1.shape, lambda bi, ti: (0, 0)),
            pl.BlockSpec(w2.shape, lambda bi, ti: (0, 0)),
            pl.BlockSpec(b2.shape, lambda bi, ti: (0, 0)),
        ],
        out_specs=pl.BlockSpec((1, TN, c2), lambda bi, ti: (bi, ti, 0)),
        out_shape=jax.ShapeDtypeStruct((b, n1, c2), F32),
        compiler_params=_CP,
    )(x1, x2T, p1, p2, w1a, w1b, b1, w2, b2)


def _sa3_body(x2_ref, p2_ref, w1a_ref, w1b_ref, b1_ref, w2_ref, b2_ref,
              w3_ref, b3_ref, out_ref):
    h = jnp.maximum(jnp.dot(x2_ref[0], w1a_ref[...], preferred_element_type=F32)
                    + jnp.dot(p2_ref[0], w1b_ref[...], preferred_element_type=F32)
                    + b1_ref[...], 0.0)
    h = jnp.maximum(jnp.dot(h, w2_ref[...], preferred_element_type=F32) + b2_ref[...], 0.0)
    h = jnp.maximum(jnp.dot(h, w3_ref[...], preferred_element_type=F32) + b3_ref[...], 0.0)
    out_ref[0] = jnp.max(h, axis=0, keepdims=True)


def _sa3(l2_xyz, l2_points, layers):
    b, s, cp = l2_points.shape
    wf1, b1 = _fuse(layers[0])
    w2, b2 = _fuse(layers[1])
    w3, b3 = _fuse(layers[2])
    w1a, w1b = wf1[:3], wf1[3:]
    c3 = w3.shape[1]
    return pl.pallas_call(
        _sa3_body,
        grid=(b,),
        in_specs=[
            pl.BlockSpec((1, s, 3), lambda bi: (bi, 0, 0)),
            pl.BlockSpec((1, s, cp), lambda bi: (bi, 0, 0)),
            pl.BlockSpec(w1a.shape, lambda bi: (0, 0)),
            pl.BlockSpec(w1b.shape, lambda bi: (0, 0)),
            pl.BlockSpec(b1.shape, lambda bi: (0, 0)),
            pl.BlockSpec(w2.shape, lambda bi: (0, 0)),
            pl.BlockSpec(b2.shape, lambda bi: (0, 0)),
            pl.BlockSpec(w3.shape, lambda bi: (0, 0)),
            pl.BlockSpec(b3.shape, lambda bi: (0, 0)),
        ],
        out_specs=pl.BlockSpec((1, 1, c3), lambda bi: (bi, 0, 0)),
        out_shape=jax.ShapeDtypeStruct((b, 1, c3), F32),
        compiler_params=_CP,
    )(l2_xyz, l2_points, w1a, w1b, b1, w2, b2, w3, b3)


def _fp3_body(p1_ref, l3_ref, w1a_ref, w1b_ref, b1_ref, w2_ref, b2_ref, out_ref):
    base = jnp.dot(l3_ref[0], w1b_ref[...], preferred_element_type=F32)
    h = jnp.maximum(jnp.dot(p1_ref[0], w1a_ref[...], preferred_element_type=F32)
                    + base + b1_ref[...], 0.0)
    out_ref[0] = jnp.maximum(jnp.dot(h, w2_ref[...], preferred_element_type=F32)
                             + b2_ref[...], 0.0)


def _fp3(l2_points, l3_points, layers):
    b, s, cp = l2_points.shape
    cl3 = l3_points.shape[2]
    wf1, b1 = _fuse(layers[0])
    w2, b2 = _fuse(layers[1])
    w1a, w1b = wf1[:cp], wf1[cp:]
    c2 = w2.shape[1]
    return pl.pallas_call(
        _fp3_body,
        grid=(b,),
        in_specs=[
            pl.BlockSpec((1, s, cp), lambda bi: (bi, 0, 0)),
            pl.BlockSpec((1, 1, cl3), lambda bi: (bi, 0, 0)),
            pl.BlockSpec(w1a.shape, lambda bi: (0, 0)),
            pl.BlockSpec(w1b.shape, lambda bi: (0, 0)),
            pl.BlockSpec(b1.shape, lambda bi: (0, 0)),
            pl.BlockSpec(w2.shape, lambda bi: (0, 0)),
            pl.BlockSpec(b2.shape, lambda bi: (0, 0)),
        ],
        out_specs=pl.BlockSpec((1, s, c2), lambda bi: (bi, 0, 0)),
        out_shape=jax.ShapeDtypeStruct((b, s, c2), F32),
        compiler_params=_CP,
    )(l2_points, l3_points, w1a, w1b, b1, w2, b2)


def _head_body(x_ref, w1_ref, b1_ref, w2_ref, b2_ref, out_ref):
    feat = jnp.maximum(jnp.dot(x_ref[0], w1_ref[...], preferred_element_type=F32)
                       + b1_ref[...], 0.0)
    logits = jnp.dot(feat, w2_ref[...], preferred_element_type=F32) + b2_ref[...]
    m = jnp.max(logits, axis=1, keepdims=True)
    sh = logits - m
    out_ref[0] = sh - jnp.log(jnp.sum(jnp.exp(sh), axis=1, keepdims=True))


def _head(l0_fp, conv1, conv2):
    b, n, _ = l0_fp.shape
    w1, b1 = _fuse(conv1)
    w2 = jnp.transpose(conv2['w'])
    b2 = conv2['b'][None, :]
    nc = w2.shape[1]
    return pl.pallas_call(
        _head_body,
        grid=(b,),
        in_specs=[
            pl.BlockSpec((1, n, l0_fp.shape[2]), lambda bi: (bi, 0, 0)),
            pl.BlockSpec(w1.shape, lambda bi: (0, 0)),
            pl.BlockSpec(b1.shape, lambda bi: (0, 0)),
            pl.BlockSpec(w2.shape, lambda bi: (0, 0)),
            pl.BlockSpec(b2.shape, lambda bi: (0, 0)),
        ],
        out_specs=pl.BlockSpec((1, n, nc), lambda bi: (bi, 0, 0)),
        out_shape=jax.ShapeDtypeStruct((b, n, nc), F32),
        compiler_params=_CP,
    )(l0_fp, w1, b1, w2, b2)


def _fuse(layer):
    w = jnp.transpose(layer['w']) * layer['gamma'][None, :]
    bias = (layer['b'] * layer['gamma'] + layer['beta'])[None, :]
    return w, bias


def kernel(xyz, cls_label, params):
    b, _, n = xyz.shape
    l0_xyz = jnp.transpose(xyz, (0, 2, 1))

    l1_xyz = _fps(l0_xyz, 1024)
    pu1 = jnp.concatenate([l0_xyz, l0_xyz], axis=-1)
    outs = []
    for radius, K, layers in zip([0.1, 0.2, 0.4], [32, 64, 128], params['sa1']):
        outs.append(_sa_branch(pu1, xyz, l1_xyz, radius, K, 256, 4, layers))
    l1_points = jnp.concatenate(outs, axis=-1)

    l2_xyz = _fps(l1_xyz, 128)
    pu2 = jnp.concatenate([l1_points, l1_xyz], axis=-1)
    xyzT1 = jnp.transpose(l1_xyz, (0, 2, 1))
    outs2 = []
    for radius, K, layers in zip([0.4, 0.8], [64, 128], params['sa2']):
        outs2.append(_sa_branch(pu2, xyzT1, l2_xyz, radius, K, 128, 8, layers))
    l2_points = jnp.concatenate(outs2, axis=-1)

    l3_points = _sa3(l2_xyz, l2_points, params['sa3'])
    l2_fp = _fp3(l2_points, l3_points, params['fp3'])
    l1_fp = _fp(l1_xyz, l2_xyz, l1_points, l2_fp, 256, params['fp2'])
    cls_one = jnp.broadcast_to(cls_label.reshape(b, 1, 1), (b, n, 1))
    p1 = jnp.concatenate([cls_one, l0_xyz, l0_xyz], axis=-1)
    l0_fp = _fp(l0_xyz, l1_xyz, p1, l1_fp, 256, params['fp1'])

    x = _head(l0_fp, params['conv1'], params['conv2'])
    return x, jnp.transpose(l3_points, (0, 2, 1))


# speedup vs baseline: 1.5749x; 1.2362x over previous
import functools

import jax
import jax.numpy as jnp
import numpy as np
from jax.experimental import pallas as pl
from jax.experimental.pallas import tpu as pltpu

F32 = jnp.float32
BF16 = jnp.bfloat16

_CP = pltpu.CompilerParams(vmem_limit_bytes=100 * 1024 * 1024)


def _fps_body(npoint, n, xc_ref, yc_ref, zc_ref, newxyz_ref):
    b = xc_ref.shape[0]
    r = n // 128
    lin = (jax.lax.broadcasted_iota(jnp.int32, (1, r, 128), 1) * 128
           + jax.lax.broadcasted_iota(jnp.int32, (1, r, 128), 2))
    xc = xc_ref[...]
    yc = yc_ref[...]
    zc = zc_ref[...]

    def red_sum(a):
        return jnp.sum(jnp.sum(a, axis=2, keepdims=True), axis=1, keepdims=True)

    def red_max(a):
        return jnp.max(jnp.max(a, axis=2, keepdims=True), axis=1, keepdims=True)

    def red_min(a):
        return jnp.min(jnp.min(a, axis=2, keepdims=True), axis=1, keepdims=True)

    def body(i, carry):
        dist, f = carry
        onehot = lin == f
        cx = red_sum(jnp.where(onehot, xc, 0.0))
        cy = red_sum(jnp.where(onehot, yc, 0.0))
        cz = red_sum(jnp.where(onehot, zc, 0.0))
        for bb in range(b):
            newxyz_ref[bb, pl.ds(i, 1), 0:1] = cx[bb]
            newxyz_ref[bb, pl.ds(i, 1), 1:2] = cy[bb]
            newxyz_ref[bb, pl.ds(i, 1), 2:3] = cz[bb]
        dx = xc - cx
        dy = yc - cy
        dz = zc - cz
        d = (dx * dx + dy * dy) + dz * dz
        dist = jnp.minimum(dist, d)
        m = red_max(dist)
        fn = red_min(jnp.where(dist >= m, lin, n))
        return dist, fn

    dist0 = jnp.full((b, r, 128), 1e10, F32)
    f0 = jnp.zeros((b, 1, 1), jnp.int32)
    jax.lax.fori_loop(0, npoint, body, (dist0, f0))


def _fps(xyz_pts, npoint):
    b, n, _ = xyz_pts.shape
    r = n // 128
    ch = jnp.transpose(xyz_pts, (0, 2, 1)).reshape(b, 3, r, 128)
    return pl.pallas_call(
        functools.partial(_fps_body, npoint, n),
        out_shape=jax.ShapeDtypeStruct((b, npoint, 3), F32),
        compiler_params=_CP,
    )(ch[:, 0], ch[:, 1], ch[:, 2])


def _sa_body(pu_ref, xyzT_ref, cxyz_ref, tri_ref, wu_ref, wvx_ref, b1_ref,
             w2_ref, b2_ref, w3_ref, b3_ref, out_ref, ucat_ref,
             *, r2, K, Kc, TS, n, C1, C3):
    t = pl.program_id(1)

    @pl.when(t == 0)
    def _():
        u = jnp.dot(pu_ref[0], wu_ref[...], preferred_element_type=F32)
        uhi = u.astype(BF16)
        ucat_ref[:, :C1] = uhi
        ucat_ref[:, C1:] = (u - uhi.astype(F32)).astype(BF16)

    cs = cxyz_ref[0]
    xT = xyzT_ref[0]
    cn = (cs[:, 0:1] * cs[:, 0:1] + cs[:, 1:2] * cs[:, 1:2]) + cs[:, 2:3] * cs[:, 2:3]
    xn = (xT[0:1] * xT[0:1] + xT[1:2] * xT[1:2]) + xT[2:3] * xT[2:3]
    dotp = jnp.dot(cs.astype(BF16), xT.astype(BF16), preferred_element_type=F32)
    d = (cn + xn) - 2.0 * dotp
    mask = d <= r2
    rank = jnp.dot(mask.astype(BF16), tri_ref[...], preferred_element_type=F32)
    count = jnp.sum(mask.astype(F32), axis=1, keepdims=True)
    rankk = jnp.where(mask, rank, -1.0)
    lastcol = jax.lax.broadcasted_iota(jnp.int32, (1, n), 1) == (n - 1)
    rankk = jnp.where(jnp.logical_and(count == 0.0, lastcol), 0.0, rankk)
    count = jnp.maximum(count, 1.0)
    vterm = b1_ref[...] - jnp.dot(cs, wvx_ref[...], preferred_element_type=F32)
    ucat = ucat_ref[...]
    w2 = w2_ref[...]
    b2 = b2_ref[...]
    w3 = w3_ref[...]
    b3 = b3_ref[...]

    nchunks = jnp.ceil(jnp.minimum(jnp.max(count), float(K)) / Kc).astype(jnp.int32)

    def chunk(kc, acc):
        kio = jax.lax.broadcasted_iota(jnp.int32, (1, Kc, 1), 1) + kc * Kc
        kiof = kio.astype(F32)
        psel = (rankk[:, None, :] == kiof).astype(BF16)
        f = jnp.dot(psel.reshape(TS * Kc, n), ucat, preferred_element_type=F32)
        f1 = f[:, :C1] + f[:, C1:]
        h = jnp.maximum(f1.reshape(TS, Kc, C1) + vterm[:, None, :], 0.0)
        h = jnp.maximum(jnp.dot(h.reshape(TS * Kc, C1), w2,
                                preferred_element_type=F32) + b2, 0.0)
        h = jnp.maximum(jnp.dot(h, w3, preferred_element_type=F32) + b3, 0.0)
        valid = kiof < count[:, :, None]
        hm = jnp.where(valid, h.reshape(TS, Kc, C3), 0.0)
        return jnp.maximum(acc, jnp.max(hm, axis=1))

    out_ref[0] = jax.lax.fori_loop(0, nchunks, chunk, jnp.zeros((TS, C3), F32))


def _sa_branch(pu, xyzT, cxyz, radius, K, TS, Kc, layers):
    b, n, cpu = pu.shape
    s = cxyz.shape[1]
    wu, b1 = _fuse(layers[0])
    w2, b2 = _fuse(layers[1])
    w3, b3 = _fuse(layers[2])
    c1, c3 = wu.shape[1], w3.shape[1]
    wvx = wu[cpu - 3:]
    tri = (jax.lax.broadcasted_iota(jnp.int32, (n, n), 0)
           < jax.lax.broadcasted_iota(jnp.int32, (n, n), 1)).astype(BF16)
    r2 = float(np.float32(radius ** 2))
    body = functools.partial(_sa_body, r2=r2, K=K, Kc=Kc, TS=TS, n=n,
                             C1=c1, C3=c3)
    return pl.pallas_call(
        body,
        grid=(b, s // TS),
        in_specs=[
            pl.BlockSpec((1, n, cpu), lambda bi, ti: (bi, 0, 0)),
            pl.BlockSpec((1, 3, n), lambda bi, ti: (bi, 0, 0)),
            pl.BlockSpec((1, TS, 3), lambda bi, ti: (bi, ti, 0)),
            pl.BlockSpec((n, n), lambda bi, ti: (0, 0)),
            pl.BlockSpec(wu.shape, lambda bi, ti: (0, 0)),
            pl.BlockSpec(wvx.shape, lambda bi, ti: (0, 0)),
            pl.BlockSpec(b1.shape, lambda bi, ti: (0, 0)),
            pl.BlockSpec(w2.shape, lambda bi, ti: (0, 0)),
            pl.BlockSpec(b2.shape, lambda bi, ti: (0, 0)),
            pl.BlockSpec(w3.shape, lambda bi, ti: (0, 0)),
            pl.BlockSpec(b3.shape, lambda bi, ti: (0, 0)),
        ],
        out_specs=pl.BlockSpec((1, TS, c3), lambda bi, ti: (bi, ti, 0)),
        out_shape=jax.ShapeDtypeStruct((b, s, c3), F32),
        scratch_shapes=[pltpu.VMEM((n, 2 * c1), BF16)],
        compiler_params=_CP,
    )(pu, xyzT, cxyz, tri, wu, wvx, b1, w2, b2, w3, b3)


def _fp_body(x1_ref, x2T_ref, p1_ref, p2_ref, w1a_ref, w1b_ref, b1_ref,
             w2_ref, b2_ref, out_ref, *, S2):
    cs = x1_ref[0]
    xT = x2T_ref[0]
    cn = (cs[:, 0:1] * cs[:, 0:1] + cs[:, 1:2] * cs[:, 1:2]) + cs[:, 2:3] * cs[:, 2:3]
    xn = (xT[0:1] * xT[0:1] + xT[1:2] * xT[1:2]) + xT[2:3] * xT[2:3]
    dotp = jnp.dot(cs.astype(BF16), xT.astype(BF16), preferred_element_type=F32)
    d = (cn + xn) - 2.0 * dotp
    io = jax.lax.broadcasted_iota(jnp.int32, (1, S2), 1)
    inf = jnp.float32(float("inf"))
    m1 = jnp.min(d, axis=1, keepdims=True)
    i1 = jnp.min(jnp.where(d <= m1, io, S2), axis=1, keepdims=True)
    d1 = jnp.where(io == i1, inf, d)
    m2 = jnp.min(d1, axis=1, keepdims=True)
    i2 = jnp.min(jnp.where(d1 <= m2, io, S2), axis=1, keepdims=True)
    d2 = jnp.where(io == i2, inf, d1)
    m3 = jnp.min(d2, axis=1, keepdims=True)
    i3 = jnp.min(jnp.where(d2 <= m3, io, S2), axis=1, keepdims=True)
    r1 = 1.0 / (m1 + 1e-8)
    r2 = 1.0 / (m2 + 1e-8)
    r3 = 1.0 / (m3 + 1e-8)
    s = (r1 + r2) + r3
    W = (jnp.where(io == i1, r1 / s, 0.0)
         + jnp.where(io == i2, r2 / s, 0.0)
         + jnp.where(io == i3, r3 / s, 0.0))
    interp = jnp.dot(W, p2_ref[0], preferred_element_type=F32)
    h = jnp.maximum(jnp.dot(p1_ref[0], w1a_ref[...], preferred_element_type=F32)
                    + jnp.dot(interp, w1b_ref[...], preferred_element_type=F32)
                    + b1_ref[...], 0.0)
    out_ref[0] = jnp.maximum(jnp.dot(h, w2_ref[...], preferred_element_type=F32)
                             + b2_ref[...], 0.0)


def _fp(x1, x2, p1, p2, TN, layers):
    b, n1, cp1 = p1.shape
    s2, cp2 = p2.shape[1], p2.shape[2]
    wf1, b1 = _fuse(layers[0])
    w2, b2 = _fuse(layers[1])
    w1a, w1b = wf1[:cp1], wf1[cp1:]
    c2 = w2.shape[1]
    x2T = jnp.transpose(x2, (0, 2, 1))
    return pl.pallas_call(
        functools.partial(_fp_body, S2=s2),
        grid=(b, n1 // TN),
        in_specs=[
            pl.BlockSpec((1, TN, 3), lambda bi, ti: (bi, ti, 0)),
            pl.BlockSpec((1, 3, s2), lambda bi, ti: (bi, 0, 0)),
            pl.BlockSpec((1, TN, cp1), lambda bi, ti: (bi, ti, 0)),
            pl.BlockSpec((1, s2, cp2), lambda bi, ti: (bi, 0, 0)),
            pl.BlockSpec(w1a.shape, lambda bi, ti: (0, 0)),
            pl.BlockSpec(w1b.shape, lambda bi, ti: (0, 0)),
            pl.BlockSpec(b1.shape, lambda bi, ti: (0, 0)),
            pl.BlockSpec(w2.shape, lambda bi, ti: (0, 0)),
            pl.BlockSpec(b2.shape, lambda bi, ti: (0, 0)),
        ],
        out_specs=pl.BlockSpec((1, TN, c2), lambda bi, ti: (bi, ti, 0)),
        out_shape=jax.ShapeDtypeStruct((b, n1, c2), F32),
        compiler_params=_CP,
    )(x1, x2T, p1, p2, w1a, w1b, b1, w2, b2)


def _sa3_body(x2_ref, p2_ref, w1a_ref, w1b_ref, b1_ref, w2_ref, b2_ref,
              w3_ref, b3_ref, out_ref):
    h = jnp.maximum(jnp.dot(x2_ref[0], w1a_ref[...], preferred_element_type=F32)
                    + jnp.dot(p2_ref[0], w1b_ref[...], preferred_element_type=F32)
                    + b1_ref[...], 0.0)
    h = jnp.maximum(jnp.dot(h, w2_ref[...], preferred_element_type=F32) + b2_ref[...], 0.0)
    h = jnp.maximum(jnp.dot(h, w3_ref[...], preferred_element_type=F32) + b3_ref[...], 0.0)
    out_ref[0] = jnp.max(h, axis=0, keepdims=True)


def _sa3(l2_xyz, l2_points, layers):
    b, s, cp = l2_points.shape
    wf1, b1 = _fuse(layers[0])
    w2, b2 = _fuse(layers[1])
    w3, b3 = _fuse(layers[2])
    w1a, w1b = wf1[:3], wf1[3:]
    c3 = w3.shape[1]
    return pl.pallas_call(
        _sa3_body,
        grid=(b,),
        in_specs=[
            pl.BlockSpec((1, s, 3), lambda bi: (bi, 0, 0)),
            pl.BlockSpec((1, s, cp), lambda bi: (bi, 0, 0)),
            pl.BlockSpec(w1a.shape, lambda bi: (0, 0)),
            pl.BlockSpec(w1b.shape, lambda bi: (0, 0)),
            pl.BlockSpec(b1.shape, lambda bi: (0, 0)),
            pl.BlockSpec(w2.shape, lambda bi: (0, 0)),
            pl.BlockSpec(b2.shape, lambda bi: (0, 0)),
            pl.BlockSpec(w3.shape, lambda bi: (0, 0)),
            pl.BlockSpec(b3.shape, lambda bi: (0, 0)),
        ],
        out_specs=pl.BlockSpec((1, 1, c3), lambda bi: (bi, 0, 0)),
        out_shape=jax.ShapeDtypeStruct((b, 1, c3), F32),
        compiler_params=_CP,
    )(l2_xyz, l2_points, w1a, w1b, b1, w2, b2, w3, b3)


def _fp3_body(p1_ref, l3_ref, w1a_ref, w1b_ref, b1_ref, w2_ref, b2_ref, out_ref):
    base = jnp.dot(l3_ref[0], w1b_ref[...], preferred_element_type=F32)
    h = jnp.maximum(jnp.dot(p1_ref[0], w1a_ref[...], preferred_element_type=F32)
                    + base + b1_ref[...], 0.0)
    out_ref[0] = jnp.maximum(jnp.dot(h, w2_ref[...], preferred_element_type=F32)
                             + b2_ref[...], 0.0)


def _fp3(l2_points, l3_points, layers):
    b, s, cp = l2_points.shape
    cl3 = l3_points.shape[2]
    wf1, b1 = _fuse(layers[0])
    w2, b2 = _fuse(layers[1])
    w1a, w1b = wf1[:cp], wf1[cp:]
    c2 = w2.shape[1]
    return pl.pallas_call(
        _fp3_body,
        grid=(b,),
        in_specs=[
            pl.BlockSpec((1, s, cp), lambda bi: (bi, 0, 0)),
            pl.BlockSpec((1, 1, cl3), lambda bi: (bi, 0, 0)),
            pl.BlockSpec(w1a.shape, lambda bi: (0, 0)),
            pl.BlockSpec(w1b.shape, lambda bi: (0, 0)),
            pl.BlockSpec(b1.shape, lambda bi: (0, 0)),
            pl.BlockSpec(w2.shape, lambda bi: (0, 0)),
            pl.BlockSpec(b2.shape, lambda bi: (0, 0)),
        ],
        out_specs=pl.BlockSpec((1, s, c2), lambda bi: (bi, 0, 0)),
        out_shape=jax.ShapeDtypeStruct((b, s, c2), F32),
        compiler_params=_CP,
    )(l2_points, l3_points, w1a, w1b, b1, w2, b2)


def _head_body(x_ref, w1_ref, b1_ref, w2_ref, b2_ref, out_ref):
    feat = jnp.maximum(jnp.dot(x_ref[0], w1_ref[...], preferred_element_type=F32)
                       + b1_ref[...], 0.0)
    logits = jnp.dot(feat, w2_ref[...], preferred_element_type=F32) + b2_ref[...]
    m = jnp.max(logits, axis=1, keepdims=True)
    sh = logits - m
    out_ref[0] = sh - jnp.log(jnp.sum(jnp.exp(sh), axis=1, keepdims=True))


def _head(l0_fp, conv1, conv2):
    b, n, _ = l0_fp.shape
    w1, b1 = _fuse(conv1)
    w2 = jnp.transpose(conv2['w'])
    b2 = conv2['b'][None, :]
    nc = w2.shape[1]
    return pl.pallas_call(
        _head_body,
        grid=(b,),
        in_specs=[
            pl.BlockSpec((1, n, l0_fp.shape[2]), lambda bi: (bi, 0, 0)),
            pl.BlockSpec(w1.shape, lambda bi: (0, 0)),
            pl.BlockSpec(b1.shape, lambda bi: (0, 0)),
            pl.BlockSpec(w2.shape, lambda bi: (0, 0)),
            pl.BlockSpec(b2.shape, lambda bi: (0, 0)),
        ],
        out_specs=pl.BlockSpec((1, n, nc), lambda bi: (bi, 0, 0)),
        out_shape=jax.ShapeDtypeStruct((b, n, nc), F32),
        compiler_params=_CP,
    )(l0_fp, w1, b1, w2, b2)


def _fuse(layer):
    w = jnp.transpose(layer['w']) * layer['gamma'][None, :]
    bias = (layer['b'] * layer['gamma'] + layer['beta'])[None, :]
    return w, bias


def kernel(xyz, cls_label, params):
    b, _, n = xyz.shape
    l0_xyz = jnp.transpose(xyz, (0, 2, 1))

    l1_xyz = _fps(l0_xyz, 1024)
    pu1 = jnp.concatenate([l0_xyz, l0_xyz], axis=-1)
    outs = []
    for radius, K, layers in zip([0.1, 0.2, 0.4], [32, 64, 128], params['sa1']):
        outs.append(_sa_branch(pu1, xyz, l1_xyz, radius, K, 256, 4, layers))
    l1_points = jnp.concatenate(outs, axis=-1)

    l2_xyz = _fps(l1_xyz, 128)
    pu2 = jnp.concatenate([l1_points, l1_xyz], axis=-1)
    xyzT1 = jnp.transpose(l1_xyz, (0, 2, 1))
    outs2 = []
    for radius, K, layers in zip([0.4, 0.8], [64, 128], params['sa2']):
        outs2.append(_sa_branch(pu2, xyzT1, l2_xyz, radius, K, 128, 8, layers))
    l2_points = jnp.concatenate(outs2, axis=-1)

    l3_points = _sa3(l2_xyz, l2_points, params['sa3'])
    l2_fp = _fp3(l2_points, l3_points, params['fp3'])
    l1_fp = _fp(l1_xyz, l2_xyz, l1_points, l2_fp, 256, params['fp2'])
    cls_one = jnp.broadcast_to(cls_label.reshape(b, 1, 1), (b, n, 1))
    p1 = jnp.concatenate([cls_one, l0_xyz, l0_xyz], axis=-1)
    l0_fp = _fp(l0_xyz, l1_xyz, p1, l1_fp, 256, params['fp1'])

    x = _head(l0_fp, params['conv1'], params['conv2'])
    return x, jnp.transpose(l3_points, (0, 2, 1))
